# baseline (speedup 1.0000x reference)
import functools

import jax
import jax.numpy as jnp
from jax import lax
from jax.experimental import pallas as pl
from jax.experimental.pallas import tpu as pltpu
from jax.experimental.pallas import tpu_sc as plsc


def _build(B, S, D):
    mesh = plsc.ScalarSubcoreMesh(axis_name="c", num_cores=1)

    @functools.partial(
        pl.kernel,
        mesh=mesh,
        out_type=jax.ShapeDtypeStruct((B, D), jnp.float32),
    )
    def body(emb_hbm, mask_hbm, out_hbm):
        for b in range(B):
            pltpu.sync_copy(emb_hbm.at[b * S + (S - 1)], out_hbm.at[b])

    return body


def kernel(token_embeddings, attention_mask):
    B, S, D = token_embeddings.shape
    emb2d = token_embeddings.reshape(B * S, D)
    return _build(B, S, D)(emb2d, attention_mask)


# speedup vs baseline: 1.0166x; 1.0166x over previous
import functools

import jax
import jax.numpy as jnp
from jax import lax
from jax.experimental import pallas as pl
from jax.experimental.pallas import tpu as pltpu
from jax.experimental.pallas import tpu_sc as plsc

_LANES = 16
_UNROLL = 8
_WPB = 4


def _build(B, S, D):
    mesh = plsc.VectorSubcoreMesh(
        core_axis_name="c", subcore_axis_name="s", num_cores=1
    )
    chunk = D // _WPB

    @functools.partial(
        pl.kernel,
        mesh=mesh,
        out_type=jax.ShapeDtypeStruct((B, D), jnp.float32),
        scratch_types=[
            pltpu.VMEM((S,), jnp.int32),
        ],
    )
    def body(emb_hbm, mask_hbm, out_hbm, mask_v):
        sid = lax.axis_index("s")

        @pl.when(sid < B * _WPB)
        def _():
            b = sid // _WPB
            q = sid % _WPB
            pltpu.sync_copy(mask_hbm.at[b], mask_v)

            span = _LANES * _UNROLL

            def step(i, accs):
                base = i * span
                return tuple(
                    a + mask_v[pl.ds(base + k * _LANES, _LANES)]
                    for k, a in enumerate(accs)
                )

            accs = lax.fori_loop(
                0, S // span, step,
                tuple(jnp.zeros((_LANES,), jnp.int32) for _ in range(_UNROLL)),
            )
            acc = accs[0]
            for a in accs[1:]:
                acc = acc + a
            total = acc[0]
            for lane in range(1, _LANES):
                total = total + acc[lane]

            idx = b * S + total - 1
            off = q * chunk
            pltpu.sync_copy(
                emb_hbm.at[idx, pl.ds(off, chunk)],
                out_hbm.at[b, pl.ds(off, chunk)],
            )

    return body


def kernel(token_embeddings, attention_mask):
    B, S, D = token_embeddings.shape
    emb2d = token_embeddings.reshape(B * S, D)
    return _build(B, S, D)(emb2d, attention_mask)
